# baseline (speedup 1.0000x reference)
import functools

import jax
import jax.numpy as jnp
from jax import lax
from jax.experimental import pallas as pl
from jax.experimental.pallas import tpu as pltpu
from jax.experimental.pallas import tpu_sc as plsc

B = 128
H = 64
N = 10000
S = 6400
SC_WORKERS = 32
SC_CHUNK = 8
U_CHUNKS = 50
CS2 = 150
R2 = 4


def _gru_body(users_ref, items_ref, um_any, im_any,
              wih_u, whh_u, bih_u, bhh_u, wih_i, whh_i, bih_i, bhh_i,
              irow, icol,
              ue_out, ie_out, slab_out,
              slab_u, slab_i, sem_g):
    for b in range(B):
        pltpu.make_async_copy(um_any.at[users_ref[b]], slab_u.at[b], sem_g).start()
        pltpu.make_async_copy(im_any.at[items_ref[b]], slab_i.at[b], sem_g).start()
    for b in range(B):
        pltpu.make_async_copy(um_any.at[users_ref[b]], slab_u.at[b], sem_g).wait()
        pltpu.make_async_copy(im_any.at[items_ref[b]], slab_i.at[b], sem_g).wait()

    eq3 = (jax.lax.broadcasted_iota(jnp.int32, (B, H, B), 0)
           == jax.lax.broadcasted_iota(jnp.int32, (B, H, B), 2))
    ueT = jnp.sum(jnp.where(eq3, slab_u[...], 0.0), axis=0)
    ieT = jnp.sum(jnp.where(eq3, slab_i[...], 0.0), axis=0)

    def gru_t(xT, hT, wih, whh, bih, bhh):
        giT = jnp.dot(wih, xT, preferred_element_type=jnp.float32) + bih
        ghT = jnp.dot(whh, hT, preferred_element_type=jnp.float32) + bhh
        r = jax.nn.sigmoid(giT[:H] + ghT[:H])
        z = jax.nn.sigmoid(giT[H:2 * H] + ghT[H:2 * H])
        nn = jnp.tanh(giT[2 * H:] + r * ghT[2 * H:])
        return (1.0 - z) * nn + z * hT

    xT_u = jnp.concatenate([ueT, ieT], axis=0)
    xT_i = jnp.concatenate([ieT, ueT], axis=0)
    nu = gru_t(xT_u, ueT, wih_u[...], whh_u[...], bih_u[...], bhh_u[...])
    ni = gru_t(xT_i, ieT, wih_i[...], whh_i[...], bih_i[...], bhh_i[...])
    ue_out[...] = nu
    ie_out[...] = ni
    mi = (icol[...] == irow[...])[:, None, :]
    slab_out[...] = jnp.where(mi, ni[None], slab_i[...])


def _sc_copy_body(im_hbm, im_o, buf):
    wid = lax.axis_index("s") * 2 + lax.axis_index("c")
    rw = S // SC_WORKERS
    base = wid * rw
    for j in range(rw // SC_CHUNK):
        pltpu.sync_copy(im_hbm.at[pl.ds(base + j * SC_CHUNK, SC_CHUNK)], buf)
        pltpu.sync_copy(buf, im_o.at[pl.ds(base + j * SC_CHUNK, SC_CHUNK)])


def _um_body(um_blk, nu, urow, umo_blk):
    c = pl.program_id(0)
    cs = um_blk.shape[0]
    iota0 = jax.lax.broadcasted_iota(jnp.int32, (cs, H, B), 0)
    locs = (urow[...] - c * cs)[None]
    umo_blk[...] = jnp.where(iota0 == locs, nu[...][None], um_blk[...])


def _im_body(items_ref, im_o_in, im_any, ni, irow, slab_p,
             imo_any, ibuf, obuf, sem_in, sem_out, sem_s):
    t = (N - S) // CS2

    def in_c(k, slot):
        return pltpu.make_async_copy(im_any.at[pl.ds(S + k * CS2, CS2)],
                                     ibuf.at[slot], sem_in.at[slot])

    def out_c(k, slot):
        return pltpu.make_async_copy(obuf.at[slot],
                                     imo_any.at[pl.ds(S + k * CS2, CS2)],
                                     sem_out.at[slot])

    for r0 in range(R2):
        in_c(r0, r0).start()

    iota0 = jax.lax.broadcasted_iota(jnp.int32, (CS2, H, B), 0)

    def step(k, carry):
        slot = lax.rem(k, R2)
        in_c(k, slot).wait()

        @pl.when(k >= R2)
        def _():
            out_c(k - R2, slot).wait()

        locs = (irow[...] - (S + k * CS2))[None]
        obuf.at[slot][...] = jnp.where(iota0 == locs, ni[...][None],
                                       ibuf.at[slot][...])
        out_c(k, slot).start()

        @pl.when(k + R2 < t)
        def _():
            in_c(k + R2, slot).start()
        return carry

    lax.fori_loop(0, t, step, 0)
    for r0 in range(R2):
        k = t - R2 + r0
        out_c(k, k % R2).wait()

    for b in range(B):
        pltpu.make_async_copy(slab_p.at[b], imo_any.at[items_ref[b]], sem_s).start()
    for b in range(B):
        pltpu.make_async_copy(slab_p.at[b], imo_any.at[items_ref[b]], sem_s).wait()


def kernel(user_memory, item_memory, users, items,
           W_ih_u, W_hh_u, b_ih_u, b_hh_u,
           W_ih_i, W_hh_i, b_ih_i, b_hh_i):
    users = users.astype(jnp.int32)
    items = items.astype(jnp.int32)
    um_t = jnp.transpose(user_memory, (1, 2, 0))
    im_t = jnp.transpose(item_memory, (1, 2, 0))

    anyspec = pl.BlockSpec(memory_space=pl.ANY)
    full = pl.BlockSpec(memory_space=pltpu.VMEM)

    gru_spec = pltpu.PrefetchScalarGridSpec(
        num_scalar_prefetch=2,
        grid=(),
        in_specs=[anyspec, anyspec] + [full] * 10,
        out_specs=[full, full, full],
        scratch_shapes=[
            pltpu.VMEM((B, H, B), jnp.float32),
            pltpu.VMEM((B, H, B), jnp.float32),
            pltpu.SemaphoreType.DMA,
        ],
    )
    ueT, ieT, slab_p = pl.pallas_call(
        _gru_body,
        grid_spec=gru_spec,
        out_shape=(
            jax.ShapeDtypeStruct((H, B), jnp.float32),
            jax.ShapeDtypeStruct((H, B), jnp.float32),
            jax.ShapeDtypeStruct((B, H, B), jnp.float32),
        ),
        name="limnet_gru",
    )(users, items, um_t, im_t,
      W_ih_u, W_hh_u, b_ih_u.reshape(3 * H, 1), b_hh_u.reshape(3 * H, 1),
      W_ih_i, W_hh_i, b_ih_i.reshape(3 * H, 1), b_hh_i.reshape(3 * H, 1),
      items.reshape(1, B), items.reshape(B, 1))

    sc_copy = functools.partial(
        pl.kernel,
        out_type=jax.ShapeDtypeStruct((N, H, B), jnp.float32),
        mesh=plsc.VectorSubcoreMesh(core_axis_name="c", subcore_axis_name="s"),
        scratch_types=[pltpu.VMEM((SC_CHUNK, H, B), jnp.float32)],
    )(_sc_copy_body)
    im_o = sc_copy(im_t)

    cs = N // U_CHUNKS
    new_um_t = pl.pallas_call(
        _um_body,
        grid=(U_CHUNKS,),
        in_specs=[pl.BlockSpec((cs, H, B), lambda c: (c, 0, 0)), full, full],
        out_specs=pl.BlockSpec((cs, H, B), lambda c: (c, 0, 0)),
        out_shape=jax.ShapeDtypeStruct((N, H, B), jnp.float32),
        compiler_params=pltpu.CompilerParams(
            dimension_semantics=("arbitrary",),
            vmem_limit_bytes=110 * 1024 * 1024),
        name="limnet_um_copy",
    )(um_t, ueT, users.reshape(1, B))

    im_spec = pltpu.PrefetchScalarGridSpec(
        num_scalar_prefetch=1,
        grid=(),
        in_specs=[anyspec, anyspec, full, full, full],
        out_specs=[anyspec],
        scratch_shapes=[
            pltpu.VMEM((R2, CS2, H, B), jnp.float32),
            pltpu.VMEM((R2, CS2, H, B), jnp.float32),
            pltpu.SemaphoreType.DMA((R2,)),
            pltpu.SemaphoreType.DMA((R2,)),
            pltpu.SemaphoreType.DMA,
        ],
    )
    new_im_t, = pl.pallas_call(
        _im_body,
        grid_spec=im_spec,
        out_shape=(jax.ShapeDtypeStruct((N, H, B), jnp.float32),),
        input_output_aliases={1: 0},
        compiler_params=pltpu.CompilerParams(
            vmem_limit_bytes=80 * 1024 * 1024),
        name="limnet_im_copy",
    )(items, im_o, im_t, ieT, items.reshape(1, B), slab_p)

    new_um = jnp.transpose(new_um_t, (2, 0, 1))
    new_im = jnp.transpose(new_im_t, (2, 0, 1))
    return (ueT.T, ieT.T, new_um, new_im)


# speedup vs baseline: 1.0725x; 1.0725x over previous
import jax
import jax.numpy as jnp
from jax.experimental import pallas as pl
from jax.experimental.pallas import tpu as pltpu

B = 128
H = 64
N_CHUNKS = 50


def _body(users_ref, items_ref, um_any, im_any, um_blk, im_blk,
          wih_u, whh_u, bih_u, bhh_u, wih_i, whh_i, bih_i, bhh_i,
          urow, irow,
          ue_out, ie_out, umo_blk, imo_blk,
          slab_u, slab_i, nu_t, ni_t, sem_g):
    c = pl.program_id(0)
    cs = um_blk.shape[0]

    @pl.when(c == 0)
    def _fire_gathers():
        for b in range(B):
            pltpu.make_async_copy(um_any.at[users_ref[b]], slab_u.at[b], sem_g).start()
            pltpu.make_async_copy(im_any.at[items_ref[b]], slab_i.at[b], sem_g).start()

    @pl.when(c == 1)
    def _compute():
        for b in range(B):
            pltpu.make_async_copy(um_any.at[users_ref[b]], slab_u.at[b], sem_g).wait()
            pltpu.make_async_copy(im_any.at[items_ref[b]], slab_i.at[b], sem_g).wait()

        eq3 = (jax.lax.broadcasted_iota(jnp.int32, (B, H, B), 0)
               == jax.lax.broadcasted_iota(jnp.int32, (B, H, B), 2))
        ueT = jnp.sum(jnp.where(eq3, slab_u[...], 0.0), axis=0)
        ieT = jnp.sum(jnp.where(eq3, slab_i[...], 0.0), axis=0)

        def gru_t(xT, hT, wih, whh, bih, bhh):
            giT = jnp.dot(wih, xT, preferred_element_type=jnp.float32) + bih
            ghT = jnp.dot(whh, hT, preferred_element_type=jnp.float32) + bhh
            r = jax.nn.sigmoid(giT[:H] + ghT[:H])
            z = jax.nn.sigmoid(giT[H:2 * H] + ghT[H:2 * H])
            nn = jnp.tanh(giT[2 * H:] + r * ghT[2 * H:])
            return (1.0 - z) * nn + z * hT

        xT_u = jnp.concatenate([ueT, ieT], axis=0)
        xT_i = jnp.concatenate([ieT, ueT], axis=0)
        nu_t[...] = gru_t(xT_u, ueT, wih_u[...], whh_u[...], bih_u[...], bhh_u[...])
        ni_t[...] = gru_t(xT_i, ieT, wih_i[...], whh_i[...], bih_i[...], bhh_i[...])
        ue_out[...] = nu_t[...]
        ie_out[...] = ni_t[...]

    @pl.when(c >= 1)
    def _patched_copy():
        cc = c - 1
        iota0 = jax.lax.broadcasted_iota(jnp.int32, (cs, H, B), 0)
        locs_u = (urow[...] - cc * cs)[None]
        locs_i = (irow[...] - cc * cs)[None]
        umo_blk[...] = jnp.where(iota0 == locs_u, nu_t[...][None], um_blk[...])
        imo_blk[...] = jnp.where(iota0 == locs_i, ni_t[...][None], im_blk[...])


def kernel(user_memory, item_memory, users, items,
           W_ih_u, W_hh_u, b_ih_u, b_hh_u,
           W_ih_i, W_hh_i, b_ih_i, b_hh_i):
    users = users.astype(jnp.int32)
    items = items.astype(jnp.int32)
    um_t = jnp.transpose(user_memory, (1, 2, 0))
    im_t = jnp.transpose(item_memory, (1, 2, 0))
    n = um_t.shape[0]
    cs = n // N_CHUNKS

    out_shape = (
        jax.ShapeDtypeStruct((H, B), jnp.float32),
        jax.ShapeDtypeStruct((H, B), jnp.float32),
        jax.ShapeDtypeStruct(um_t.shape, jnp.float32),
        jax.ShapeDtypeStruct(im_t.shape, jnp.float32),
    )

    def chunk_map(c, *_):
        return (jnp.maximum(c - 1, 0), 0, 0)

    blk = pl.BlockSpec((cs, H, B), chunk_map)
    rep = pl.BlockSpec((H, B), lambda c, *_: (0, 0))
    full = pl.BlockSpec(memory_space=pltpu.VMEM)
    grid_spec = pltpu.PrefetchScalarGridSpec(
        num_scalar_prefetch=2,
        grid=(N_CHUNKS + 1,),
        in_specs=[pl.BlockSpec(memory_space=pl.ANY),
                  pl.BlockSpec(memory_space=pl.ANY),
                  blk, blk] + [full] * 10,
        out_specs=[rep, rep, blk, blk],
        scratch_shapes=[
            pltpu.VMEM((B, H, B), jnp.float32),
            pltpu.VMEM((B, H, B), jnp.float32),
            pltpu.VMEM((H, B), jnp.float32),
            pltpu.VMEM((H, B), jnp.float32),
            pltpu.SemaphoreType.DMA,
        ],
    )
    ueT, ieT, new_um_t, new_im_t = pl.pallas_call(
        _body,
        grid_spec=grid_spec,
        out_shape=out_shape,
        compiler_params=pltpu.CompilerParams(
            dimension_semantics=("arbitrary",),
            vmem_limit_bytes=110 * 1024 * 1024),
        name="limnet_step",
    )(users, items, um_t, im_t, um_t, im_t,
      W_ih_u, W_hh_u, b_ih_u.reshape(3 * H, 1), b_hh_u.reshape(3 * H, 1),
      W_ih_i, W_hh_i, b_ih_i.reshape(3 * H, 1), b_hh_i.reshape(3 * H, 1),
      users.reshape(1, B), items.reshape(1, B))
    new_um = jnp.transpose(new_um_t, (2, 0, 1))
    new_im = jnp.transpose(new_im_t, (2, 0, 1))
    return (ueT.T, ieT.T, new_um, new_im)
